# Initial kernel scaffold; baseline (speedup 1.0000x reference)
import functools
import jax
import jax.numpy as jnp
from jax import lax
from jax.experimental import pallas as pl
from jax.experimental.pallas import tpu as pltpu
from jax.experimental.pallas import tpu_sc as plsc

N = 10000
E = 320000
D = 128
H = 128

NC = 2
NS = 16
NW = NC * NS

CHUNK = 128
EW = 10112
NCHUNKS = EW // CHUNK
EPT = NW * EW
NPAD = 10240
RPT = NPAD // NS

_mesh = plsc.VectorSubcoreMesh(core_axis_name="c", subcore_axis_name="s")


def _zero_rows(buf, nrows):
    z = jnp.zeros((16,), jnp.float32)
    for i in range(nrows):
        for j in range(buf.shape[-1] // 16):
            buf[i, pl.ds(j * 16, 16)] = z


@functools.partial(
    pl.kernel,
    out_type=jax.ShapeDtypeStruct((6, NPAD), jnp.float32),
    mesh=_mesh,
    scratch_types=[
        pltpu.VMEM_SHARED((NPAD,), jnp.float32),
        pltpu.VMEM_SHARED((NPAD,), jnp.float32),
        pltpu.VMEM_SHARED((NPAD,), jnp.float32),
        pltpu.VMEM((CHUNK,), jnp.int32),
        pltpu.VMEM((CHUNK,), jnp.float32),
        pltpu.VMEM((CHUNK,), jnp.float32),
    ],
)
def _deg_kernel(cols0, w0, cols1, w1, cols2, w2, degp, d0, d1, d2, cols_v, w_v, zbuf):
    c = lax.axis_index("c")
    s = lax.axis_index("s")
    wid = s * NC + c
    z = jnp.zeros((16,), jnp.float32)
    for j in range(CHUNK // 16):
        zbuf[pl.ds(j * 16, 16)] = z
    for dk in (d0, d1, d2):
        for j in range(RPT // CHUNK):
            pltpu.sync_copy(zbuf, dk.at[pl.ds(s * RPT + j * CHUNK, CHUNK)])
    plsc.subcore_barrier()
    for cols, w, dk in ((cols0, w0, d0), (cols1, w1, d1), (cols2, w2, d2)):
        def chunk_body(ci, _, cols=cols, w=w, dk=dk):
            off = wid * EW + ci * CHUNK
            pltpu.sync_copy(cols.at[pl.ds(off, CHUNK)], cols_v)
            pltpu.sync_copy(w.at[pl.ds(off, CHUNK)], w_v)
            pltpu.sync_copy(w_v, dk.at[cols_v], add=True)
            return ()
        lax.fori_loop(0, NCHUNKS, chunk_body, ())
    plsc.subcore_barrier()
    for k, dk in enumerate((d0, d1, d2)):
        pltpu.sync_copy(dk.at[pl.ds(s * RPT, RPT)],
                        degp.at[c * 3 + k, pl.ds(s * RPT, RPT)])


@functools.partial(
    pl.kernel,
    out_type=jax.ShapeDtypeStruct((2, NPAD, H), jnp.float32),
    mesh=_mesh,
    scratch_types=[
        pltpu.VMEM_SHARED((NPAD, H), jnp.float32),
        pltpu.VMEM((NPAD,), jnp.float32),
        pltpu.VMEM((CHUNK, H), jnp.float32),
        pltpu.VMEM((CHUNK,), jnp.int32),
        pltpu.VMEM((CHUNK,), jnp.int32),
        pltpu.VMEM((CHUNK,), jnp.float32),
        pltpu.VMEM((CHUNK,), jnp.float32),
        pltpu.SemaphoreType.DMA,
    ],
)
def _agg_kernel(xw, dis, r0, c0, w0, r1, c1, w1, r2, c2, w2,
                out, acc, dis_v, rowbuf, rows_v, cols_v, w_v, norm_v, sem):
    c = lax.axis_index("c")
    s = lax.axis_index("s")
    wid = s * NC + c
    _zero_rows(rowbuf, CHUNK)
    for j in range(RPT // CHUNK):
        pltpu.sync_copy(rowbuf, acc.at[pl.ds(s * RPT + j * CHUNK, CHUNK)])
    plsc.subcore_barrier()

    for k, (rows, cols, w) in enumerate(((r0, c0, w0), (r1, c1, w1), (r2, c2, w2))):
        pltpu.sync_copy(dis.at[k], dis_v)

        def chunk_body(ci, _, rows=rows, cols=cols, w=w):
            off = wid * EW + ci * CHUNK
            pltpu.sync_copy(rows.at[pl.ds(off, CHUNK)], rows_v)
            pltpu.sync_copy(cols.at[pl.ds(off, CHUNK)], cols_v)
            pltpu.sync_copy(w.at[pl.ds(off, CHUNK)], w_v)
            gather = pltpu.async_copy(xw.at[rows_v], rowbuf, sem)
            for g in range(CHUNK // 16):
                r16 = rows_v[pl.ds(g * 16, 16)]
                c16 = cols_v[pl.ds(g * 16, 16)]
                dr = plsc.load_gather(dis_v, [r16])
                dc = plsc.load_gather(dis_v, [c16])
                norm_v[pl.ds(g * 16, 16)] = dr * w_v[pl.ds(g * 16, 16)] * dc
            gather.wait()

            def scale_body(e, _):
                nb = plsc.load_gather(norm_v, [jnp.zeros((16,), jnp.int32) + e])
                for j in range(H // 16):
                    rowbuf[e, pl.ds(j * 16, 16)] = rowbuf[e, pl.ds(j * 16, 16)] * nb
                return ()
            lax.fori_loop(0, CHUNK, scale_body, ())
            pltpu.sync_copy(rowbuf, acc.at[cols_v], add=True)
            return ()
        lax.fori_loop(0, NCHUNKS, chunk_body, ())

    plsc.subcore_barrier()
    for j in range(RPT // CHUNK):
        pltpu.sync_copy(acc.at[pl.ds(s * RPT + j * CHUNK, CHUNK)],
                        out.at[c, pl.ds(s * RPT + j * CHUNK, CHUNK)])


def _mm_body(x_ref, w_ref, o_ref):
    o_ref[...] = jnp.dot(x_ref[...], w_ref[...],
                         preferred_element_type=jnp.float32)


def _dis_body(degp_ref, dis_ref):
    deg = degp_ref[0:3, :] + degp_ref[3:6, :]
    safe = jnp.where(deg > 0, deg, 1.0)
    dis_ref[...] = jnp.where(deg > 0, lax.rsqrt(safe), 0.0)


def _final_body(p0_ref, p1_ref, b_ref, o_ref):
    s = p0_ref[0] + p1_ref[0] + 3.0 * b_ref[...]
    o_ref[...] = jnp.maximum(s, 0.0)


def _pad_edges(ei, ew):
    pad = EPT - E
    padcols = (N + (jnp.arange(pad, dtype=jnp.int32) % 16)).astype(jnp.int32)
    rows = jnp.concatenate([ei[0], jnp.zeros((pad,), jnp.int32)])
    cols = jnp.concatenate([ei[1], padcols])
    w = jnp.concatenate([ew, jnp.zeros((pad,), jnp.float32)])
    return rows, cols, w


@jax.jit
def kernel(x, edge_index0, edge_weight0, edge_index1, edge_weight1,
           edge_index2, edge_weight2, W, b):
    r0, c0, w0 = _pad_edges(edge_index0, edge_weight0)
    r1, c1, w1 = _pad_edges(edge_index1, edge_weight1)
    r2, c2, w2 = _pad_edges(edge_index2, edge_weight2)

    degp = _deg_kernel(c0, w0, c1, w1, c2, w2)

    xw = pl.pallas_call(
        _mm_body,
        out_shape=jax.ShapeDtypeStruct((N, H), jnp.float32),
        grid=(10,),
        in_specs=[pl.BlockSpec((1000, D), lambda i: (i, 0)),
                  pl.BlockSpec((D, H), lambda i: (0, 0))],
        out_specs=pl.BlockSpec((1000, H), lambda i: (i, 0)),
    )(x, W)

    dis = pl.pallas_call(
        _dis_body,
        out_shape=jax.ShapeDtypeStruct((3, NPAD), jnp.float32),
        in_specs=[pl.BlockSpec((6, NPAD), lambda: (0, 0))],
        out_specs=pl.BlockSpec((3, NPAD), lambda: (0, 0)),
    )(degp)

    p = _agg_kernel(xw, dis, r0, c0, w0, r1, c1, w1, r2, c2, w2)

    b2 = b.reshape(1, H)
    out = pl.pallas_call(
        _final_body,
        out_shape=jax.ShapeDtypeStruct((N, H), jnp.float32),
        grid=(10,),
        in_specs=[pl.BlockSpec((1, 1000, H), lambda i: (0, i, 0)),
                  pl.BlockSpec((1, 1000, H), lambda i: (1, i, 0)),
                  pl.BlockSpec((1, H), lambda i: (0, 0))],
        out_specs=pl.BlockSpec((1000, H), lambda i: (i, 0)),
    )(p, p, b2)
    return out


# speedup vs baseline: 9.9526x; 9.9526x over previous
import functools
import jax
import jax.numpy as jnp
from jax import lax
from jax.experimental import pallas as pl
from jax.experimental.pallas import tpu as pltpu
from jax.experimental.pallas import tpu_sc as plsc

N = 10000
E = 320000
D = 128
H = 128

NC = 2
NS = 16
NW = NC * NS

CHUNK = 128
EW = 10112
NCHUNKS = EW // CHUNK
EPT = NW * EW
NPAD = 10240
RPT = NPAD // NS

_sc_params = pltpu.CompilerParams(needs_layout_passes=False)


def _zero_rows(buf, nrows):
    z = jnp.zeros((16,), jnp.float32)
    for i in range(nrows):
        for j in range(buf.shape[-1] // 16):
            buf[i, pl.ds(j * 16, 16)] = z


@functools.cache
def _sc_kernels():
  _mesh = plsc.VectorSubcoreMesh(core_axis_name="c", subcore_axis_name="s",
                                 num_cores=NC, num_subcores=NS)

  @functools.partial(
      pl.kernel,
    out_type=jax.ShapeDtypeStruct((6 * NPAD,), jnp.float32),
    mesh=_mesh,
    compiler_params=_sc_params,
    scratch_types=[
        pltpu.VMEM_SHARED((NPAD,), jnp.float32),
        pltpu.VMEM_SHARED((NPAD,), jnp.float32),
        pltpu.VMEM_SHARED((NPAD,), jnp.float32),
        pltpu.VMEM((CHUNK,), jnp.int32),
        pltpu.VMEM((CHUNK,), jnp.float32),
        pltpu.VMEM((CHUNK,), jnp.float32),
        pltpu.VMEM((RPT,), jnp.float32),
    ],
  )
  def _deg_kernel(cols0, w0, cols1, w1, cols2, w2, degp, d0, d1, d2, cols_v, w_v, zbuf, vbuf):
    c = lax.axis_index("c")
    s = lax.axis_index("s")
    wid = s * NC + c
    z = jnp.zeros((16,), jnp.float32)
    for j in range(CHUNK // 16):
        zbuf[pl.ds(j * 16, 16)] = z
    for dk in (d0, d1, d2):
        for j in range(RPT // CHUNK):
            pltpu.sync_copy(zbuf, dk.at[pl.ds(s * RPT + j * CHUNK, CHUNK)])
    plsc.subcore_barrier()
    for cols, w, dk in ((cols0, w0, d0), (cols1, w1, d1), (cols2, w2, d2)):
        def chunk_body(ci, _, cols=cols, w=w, dk=dk):
            off = wid * EW + ci * CHUNK
            pltpu.sync_copy(cols.at[pl.ds(off, CHUNK)], cols_v)
            pltpu.sync_copy(w.at[pl.ds(off, CHUNK)], w_v)
            pltpu.sync_copy(w_v, dk.at[cols_v], add=True)
            return ()
        lax.fori_loop(0, NCHUNKS, chunk_body, ())
    plsc.subcore_barrier()
    for k, dk in enumerate((d0, d1, d2)):
        pltpu.sync_copy(dk.at[pl.ds(s * RPT, RPT)], vbuf)
        pltpu.sync_copy(vbuf, degp.at[pl.ds((c * 3 + k) * NPAD + s * RPT, RPT)])


  @functools.partial(
      pl.kernel,
    out_type=jax.ShapeDtypeStruct((2, NPAD, H), jnp.float32),
    mesh=_mesh,
    compiler_params=_sc_params,
    scratch_types=[
        pltpu.VMEM_SHARED((NPAD, H), jnp.float32),
        pltpu.VMEM((NPAD,), jnp.float32),
        pltpu.VMEM((CHUNK, H), jnp.float32),
        pltpu.VMEM((CHUNK,), jnp.int32),
        pltpu.VMEM((CHUNK,), jnp.int32),
        pltpu.VMEM((CHUNK,), jnp.float32),
        pltpu.VMEM((CHUNK,), jnp.float32),
        pltpu.SemaphoreType.DMA,
    ],
  )
  def _agg_kernel(xw, dis, r0, c0, w0, r1, c1, w1, r2, c2, w2,
                out, acc, dis_v, rowbuf, rows_v, cols_v, w_v, norm_v, sem):
    c = lax.axis_index("c")
    s = lax.axis_index("s")
    wid = s * NC + c
    _zero_rows(rowbuf, CHUNK)
    for j in range(RPT // CHUNK):
        pltpu.sync_copy(rowbuf, acc.at[pl.ds(s * RPT + j * CHUNK, CHUNK)])
    plsc.subcore_barrier()

    for k, (rows, cols, w) in enumerate(((r0, c0, w0), (r1, c1, w1), (r2, c2, w2))):
        pltpu.sync_copy(dis.at[pl.ds(k * NPAD, NPAD)], dis_v)

        def chunk_body(ci, _, rows=rows, cols=cols, w=w):
            off = wid * EW + ci * CHUNK
            pltpu.sync_copy(rows.at[pl.ds(off, CHUNK)], rows_v)
            pltpu.sync_copy(cols.at[pl.ds(off, CHUNK)], cols_v)
            pltpu.sync_copy(w.at[pl.ds(off, CHUNK)], w_v)
            gather = pltpu.async_copy(xw.at[rows_v], rowbuf, sem)
            for g in range(CHUNK // 16):
                r16 = rows_v[pl.ds(g * 16, 16)]
                c16 = cols_v[pl.ds(g * 16, 16)]
                dr = plsc.load_gather(dis_v, [r16])
                dc = plsc.load_gather(dis_v, [c16])
                norm_v[pl.ds(g * 16, 16)] = dr * w_v[pl.ds(g * 16, 16)] * dc
            gather.wait()

            def scale_body(e, _):
                nb = plsc.load_gather(norm_v, [jnp.zeros((16,), jnp.int32) + e])
                for j in range(H // 16):
                    rowbuf[e, pl.ds(j * 16, 16)] = rowbuf[e, pl.ds(j * 16, 16)] * nb
                return ()
            lax.fori_loop(0, CHUNK, scale_body, ())
            pltpu.sync_copy(rowbuf, acc.at[cols_v], add=True)
            return ()
        lax.fori_loop(0, NCHUNKS, chunk_body, ())

    plsc.subcore_barrier()
    for j in range(RPT // CHUNK):
        pltpu.sync_copy(acc.at[pl.ds(s * RPT + j * CHUNK, CHUNK)],
                        out.at[c, pl.ds(s * RPT + j * CHUNK, CHUNK)])


  return _deg_kernel, _agg_kernel


def _mm_body(x_ref, w_ref, o_ref):
    o_ref[...] = jnp.dot(x_ref[...], w_ref[...],
                         preferred_element_type=jnp.float32)


def _dis_body(degp_ref, dis_ref):
    deg = degp_ref[0:3, :] + degp_ref[3:6, :]
    safe = jnp.where(deg > 0, deg, 1.0)
    dis_ref[...] = jnp.where(deg > 0, lax.rsqrt(safe), 0.0)


def _final_body(p0_ref, p1_ref, b_ref, o_ref):
    s = p0_ref[0] + p1_ref[0] + 3.0 * b_ref[...]
    o_ref[...] = jnp.maximum(s, 0.0)


def _pad_edges(ei, ew):
    pad = EPT - E
    padcols = (N + (jnp.arange(pad, dtype=jnp.int32) % 16)).astype(jnp.int32)
    rows = jnp.concatenate([ei[0], jnp.zeros((pad,), jnp.int32)])
    cols = jnp.concatenate([ei[1], padcols])
    w = jnp.concatenate([ew, jnp.zeros((pad,), jnp.float32)])
    return rows, cols, w


@jax.jit
def kernel(x, edge_index0, edge_weight0, edge_index1, edge_weight1,
           edge_index2, edge_weight2, W, b):
    r0, c0, w0 = _pad_edges(edge_index0, edge_weight0)
    r1, c1, w1 = _pad_edges(edge_index1, edge_weight1)
    r2, c2, w2 = _pad_edges(edge_index2, edge_weight2)

    deg_kernel, agg_kernel = _sc_kernels()
    degp = deg_kernel(c0, w0, c1, w1, c2, w2).reshape(6, NPAD)

    xw = pl.pallas_call(
        _mm_body,
        out_shape=jax.ShapeDtypeStruct((N, H), jnp.float32),
        grid=(10,),
        in_specs=[pl.BlockSpec((1000, D), lambda i: (i, 0)),
                  pl.BlockSpec((D, H), lambda i: (0, 0))],
        out_specs=pl.BlockSpec((1000, H), lambda i: (i, 0)),
    )(x, W)

    dis = pl.pallas_call(
        _dis_body,
        out_shape=jax.ShapeDtypeStruct((3, NPAD), jnp.float32),
        in_specs=[pl.BlockSpec((6, NPAD), lambda: (0, 0))],
        out_specs=pl.BlockSpec((3, NPAD), lambda: (0, 0)),
    )(degp)

    p = agg_kernel(xw, dis.reshape(3 * NPAD), r0, c0, w0, r1, c1, w1, r2, c2, w2)

    b2 = b.reshape(1, H)
    out = pl.pallas_call(
        _final_body,
        out_shape=jax.ShapeDtypeStruct((N, H), jnp.float32),
        grid=(10,),
        in_specs=[pl.BlockSpec((1, 1000, H), lambda i: (0, i, 0)),
                  pl.BlockSpec((1, 1000, H), lambda i: (1, i, 0)),
                  pl.BlockSpec((1, H), lambda i: (0, 0))],
        out_specs=pl.BlockSpec((1000, H), lambda i: (i, 0)),
    )(p, p, b2)
    return out


# speedup vs baseline: 9.9973x; 1.0045x over previous
import functools
import jax
import jax.numpy as jnp
from jax import lax
from jax.experimental import pallas as pl
from jax.experimental.pallas import tpu as pltpu
from jax.experimental.pallas import tpu_sc as plsc

N = 10000
E = 320000
D = 128
H = 128

NC = 2
NS = 16
NW = NC * NS

CHUNK = 128
NCH = 80
EW = NCH * CHUNK
EPT = NW * EW
NPAD = 10240
RPT = NPAD // NS
G = 16

_sc_params = pltpu.CompilerParams(needs_layout_passes=False)


@functools.cache
def _sc_kernels():
  mesh = plsc.VectorSubcoreMesh(core_axis_name="c", subcore_axis_name="s",
                                num_cores=NC, num_subcores=NS)

  @functools.partial(
      pl.kernel,
      out_type=jax.ShapeDtypeStruct((6 * NPAD,), jnp.float32),
      mesh=mesh,
      compiler_params=_sc_params,
      scratch_types=[
          pltpu.VMEM_SHARED((NPAD,), jnp.float32),
          pltpu.VMEM_SHARED((NPAD,), jnp.float32),
          pltpu.VMEM_SHARED((NPAD,), jnp.float32),
          pltpu.VMEM((NCH, CHUNK), jnp.int32),
          pltpu.VMEM((NCH, CHUNK), jnp.float32),
          pltpu.VMEM((CHUNK,), jnp.float32),
          pltpu.VMEM((RPT,), jnp.float32),
      ],
  )
  def _deg_kernel(cols0, w0, cols1, w1, cols2, w2, degp,
                  d0, d1, d2, cols_a, w_a, zbuf, vbuf):
    c = lax.axis_index("c")
    s = lax.axis_index("s")
    wid = s * NC + c
    z = jnp.zeros((16,), jnp.float32)
    for j in range(CHUNK // 16):
        zbuf[pl.ds(j * 16, 16)] = z
    for dk in (d0, d1, d2):
        for j in range(RPT // CHUNK):
            pltpu.sync_copy(zbuf, dk.at[pl.ds(s * RPT + j * CHUNK, CHUNK)])
    plsc.subcore_barrier()
    for cols, w, dk in ((cols0, w0, d0), (cols1, w1, d1), (cols2, w2, d2)):
        pltpu.sync_copy(cols.at[pl.ds(wid * NCH, NCH)], cols_a)
        pltpu.sync_copy(w.at[pl.ds(wid * NCH, NCH)], w_a)

        def chunk_body(ci, _, dk=dk):
            pltpu.sync_copy(w_a.at[ci], dk.at[cols_a.at[ci]], add=True)
            return ()
        lax.fori_loop(0, NCH, chunk_body, ())
    plsc.subcore_barrier()
    for k, dk in enumerate((d0, d1, d2)):
        pltpu.sync_copy(dk.at[pl.ds(s * RPT, RPT)], vbuf)
        pltpu.sync_copy(vbuf, degp.at[pl.ds((c * 3 + k) * NPAD + s * RPT, RPT)])

  @functools.partial(
      pl.kernel,
      out_type=jax.ShapeDtypeStruct((3 * NW * NCH, CHUNK), jnp.float32),
      mesh=mesh,
      compiler_params=_sc_params,
      scratch_types=[
          pltpu.VMEM((NPAD,), jnp.float32),
          pltpu.VMEM((NCH, CHUNK), jnp.int32),
          pltpu.VMEM((NCH, CHUNK), jnp.int32),
          pltpu.VMEM((NCH, CHUNK), jnp.float32),
          pltpu.VMEM((NCH, CHUNK), jnp.float32),
      ],
  )
  def _norm_kernel(dis, r0, c0, w0, r1, c1, w1, r2, c2, w2,
                   normh, dis_v, rows_a, cols_a, w_a, norm_a):
    c = lax.axis_index("c")
    s = lax.axis_index("s")
    wid = s * NC + c
    for k, (rows, cols, w) in enumerate(((r0, c0, w0), (r1, c1, w1), (r2, c2, w2))):
        pltpu.sync_copy(dis.at[pl.ds(k * NPAD, NPAD)], dis_v)
        pltpu.sync_copy(rows.at[pl.ds(wid * NCH, NCH)], rows_a)
        pltpu.sync_copy(cols.at[pl.ds(wid * NCH, NCH)], cols_a)
        pltpu.sync_copy(w.at[pl.ds(wid * NCH, NCH)], w_a)

        def norm_body(j, _):
            for g in range(CHUNK // 16):
                r16 = rows_a[j, pl.ds(g * 16, 16)]
                c16 = cols_a[j, pl.ds(g * 16, 16)]
                dr = plsc.load_gather(dis_v, [r16])
                dc = plsc.load_gather(dis_v, [c16])
                norm_a[j, pl.ds(g * 16, 16)] = dr * w_a[j, pl.ds(g * 16, 16)] * dc
            return ()
        lax.fori_loop(0, NCH, norm_body, ())
        pltpu.sync_copy(norm_a, normh.at[pl.ds((k * NW + wid) * NCH, NCH)])

  @functools.partial(
      pl.kernel,
      out_type=jax.ShapeDtypeStruct((2, NPAD, H), jnp.float32),
      mesh=mesh,
      compiler_params=_sc_params,
      scratch_types=[
          pltpu.VMEM_SHARED((NPAD, H), jnp.float32),
          pltpu.VMEM((CHUNK, H), jnp.float32),
          pltpu.VMEM((CHUNK, H), jnp.float32),
          pltpu.VMEM((G, CHUNK), jnp.int32),
          pltpu.VMEM((G, CHUNK), jnp.int32),
          pltpu.VMEM((G, CHUNK), jnp.float32),
          pltpu.SemaphoreType.DMA,
          pltpu.SemaphoreType.DMA,
          pltpu.SemaphoreType.DMA,
          pltpu.SemaphoreType.DMA,
      ],
  )
  def _agg_kernel(xw, normh, r0, c0, r1, c1, r2, c2,
                  out, acc, bufx, bufy, rows_a, cols_a, norm_a,
                  gsx, gsy, ssx, ssy):
    c = lax.axis_index("c")
    s = lax.axis_index("s")
    wid = s * NC + c
    z = jnp.zeros((16,), jnp.float32)
    for i in range(CHUNK):
        for j in range(H // 16):
            bufx[i, pl.ds(j * 16, 16)] = z
    for j in range(RPT // CHUNK):
        pltpu.sync_copy(bufx, acc.at[pl.ds(s * RPT + j * CHUNK, CHUNK)])
    plsc.subcore_barrier()

    def scale(buf, ci):
        def scale_body(e, _):
            nb = plsc.load_gather(
                norm_a, [jnp.zeros((16,), jnp.int32) + ci,
                         jnp.zeros((16,), jnp.int32) + e])
            for j in range(H // 16):
                buf[e, pl.ds(j * 16, 16)] = buf[e, pl.ds(j * 16, 16)] * nb
            return ()
        lax.fori_loop(0, CHUNK, scale_body, ())

    for k, (rows, cols) in enumerate(((r0, c0), (r1, c1), (r2, c2))):
        def group_body(g, _, rows=rows, cols=cols, k=k):
            base = wid * NCH + g * G
            pltpu.sync_copy(rows.at[pl.ds(base, G)], rows_a)
            pltpu.sync_copy(cols.at[pl.ds(base, G)], cols_a)
            pltpu.sync_copy(normh.at[pl.ds(k * NW * NCH + base, G)], norm_a)
            pltpu.async_copy(xw.at[rows_a.at[0]], bufx, gsx)
            pltpu.async_copy(xw.at[rows_a.at[1]], bufy, gsy)

            def pair_body(i, _):
                ci = 2 * i
                pltpu.make_async_copy(xw.at[rows_a.at[ci]], bufx, gsx).wait()
                scale(bufx, ci)
                pltpu.make_async_copy(xw.at[rows_a.at[ci + 1]], bufy, gsy).wait()
                pltpu.async_copy(bufx, acc.at[cols_a.at[ci]], ssx, add=True)
                scale(bufy, ci + 1)
                pltpu.async_copy(bufy, acc.at[cols_a.at[ci + 1]], ssy, add=True)

                @pl.when(i < G // 2 - 1)
                def _prefetch():
                    pltpu.make_async_copy(bufx, acc.at[cols_a.at[ci]], ssx).wait()
                    pltpu.async_copy(xw.at[rows_a.at[ci + 2]], bufx, gsx)
                    pltpu.make_async_copy(bufy, acc.at[cols_a.at[ci + 1]], ssy).wait()
                    pltpu.async_copy(xw.at[rows_a.at[ci + 3]], bufy, gsy)
                return ()
            lax.fori_loop(0, G // 2, pair_body, ())
            pltpu.make_async_copy(bufx, acc.at[cols_a.at[G - 2]], ssx).wait()
            pltpu.make_async_copy(bufy, acc.at[cols_a.at[G - 1]], ssy).wait()
            return ()
        lax.fori_loop(0, NCH // G, group_body, ())

    plsc.subcore_barrier()
    for j in range(RPT // CHUNK):
        pltpu.sync_copy(acc.at[pl.ds(s * RPT + j * CHUNK, CHUNK)],
                        out.at[c, pl.ds(s * RPT + j * CHUNK, CHUNK)])

  return _deg_kernel, _norm_kernel, _agg_kernel


def _mm_body(x_ref, w_ref, o_ref):
    o_ref[...] = jnp.dot(x_ref[...], w_ref[...],
                         preferred_element_type=jnp.float32)


def _dis_body(degp_ref, dis_ref):
    deg = degp_ref[0:3, :] + degp_ref[3:6, :]
    safe = jnp.where(deg > 0, deg, 1.0)
    dis_ref[...] = jnp.where(deg > 0, lax.rsqrt(safe), 0.0)


def _final_body(p0_ref, p1_ref, b_ref, o_ref):
    s = p0_ref[0] + p1_ref[0] + 3.0 * b_ref[...]
    o_ref[...] = jnp.maximum(s, 0.0)


def _pad_edges(ei, ew):
    pad = EPT - E
    padcols = (N + (jnp.arange(pad, dtype=jnp.int32) % 16)).astype(jnp.int32)
    rows = jnp.concatenate([ei[0], jnp.zeros((pad,), jnp.int32)])
    cols = jnp.concatenate([ei[1], padcols])
    w = jnp.concatenate([ew, jnp.zeros((pad,), jnp.float32)])
    return (rows.reshape(NW * NCH, CHUNK), cols.reshape(NW * NCH, CHUNK),
            w.reshape(NW * NCH, CHUNK))


@jax.jit
def kernel(x, edge_index0, edge_weight0, edge_index1, edge_weight1,
           edge_index2, edge_weight2, W, b):
    r0, c0, w0 = _pad_edges(edge_index0, edge_weight0)
    r1, c1, w1 = _pad_edges(edge_index1, edge_weight1)
    r2, c2, w2 = _pad_edges(edge_index2, edge_weight2)

    deg_kernel, norm_kernel, agg_kernel = _sc_kernels()
    degp = deg_kernel(c0, w0, c1, w1, c2, w2).reshape(6, NPAD)

    xw = pl.pallas_call(
        _mm_body,
        out_shape=jax.ShapeDtypeStruct((N, H), jnp.float32),
        grid=(10,),
        in_specs=[pl.BlockSpec((1000, D), lambda i: (i, 0)),
                  pl.BlockSpec((D, H), lambda i: (0, 0))],
        out_specs=pl.BlockSpec((1000, H), lambda i: (i, 0)),
    )(x, W)

    dis = pl.pallas_call(
        _dis_body,
        out_shape=jax.ShapeDtypeStruct((3, NPAD), jnp.float32),
        in_specs=[pl.BlockSpec((6, NPAD), lambda: (0, 0))],
        out_specs=pl.BlockSpec((3, NPAD), lambda: (0, 0)),
    )(degp)

    normh = norm_kernel(dis.reshape(3 * NPAD), r0, c0, w0, r1, c1, w1, r2, c2, w2)
    p = agg_kernel(xw, normh, r0, c0, r1, c1, r2, c2)

    b2 = b.reshape(1, H)
    out = pl.pallas_call(
        _final_body,
        out_shape=jax.ShapeDtypeStruct((N, H), jnp.float32),
        grid=(10,),
        in_specs=[pl.BlockSpec((1, 1000, H), lambda i: (0, i, 0)),
                  pl.BlockSpec((1, 1000, H), lambda i: (1, i, 0)),
                  pl.BlockSpec((1, H), lambda i: (0, 0))],
        out_specs=pl.BlockSpec((1000, H), lambda i: (i, 0)),
    )(p, p, b2)
    return out


# speedup vs baseline: 11.2737x; 1.1277x over previous
import functools
import jax
import jax.numpy as jnp
from jax import lax
from jax.experimental import pallas as pl
from jax.experimental.pallas import tpu as pltpu
from jax.experimental.pallas import tpu_sc as plsc

N = 10000
E = 320000
D = 128
H = 128

NC = 2
NS = 16
NW = NC * NS

CHUNK = 128
NCH = 80
EW = NCH * CHUNK
EPT = NW * EW
NPAD = 10240
RPT = NPAD // NS
G = 16

_sc_params = pltpu.CompilerParams(needs_layout_passes=False)


@functools.cache
def _sc_kernels():
  mesh = plsc.VectorSubcoreMesh(core_axis_name="c", subcore_axis_name="s",
                                num_cores=NC, num_subcores=NS)

  @functools.partial(
      pl.kernel,
      out_type=jax.ShapeDtypeStruct((6 * NPAD,), jnp.float32),
      mesh=mesh,
      compiler_params=_sc_params,
      scratch_types=[
          pltpu.VMEM_SHARED((NPAD,), jnp.float32),
          pltpu.VMEM_SHARED((NPAD,), jnp.float32),
          pltpu.VMEM_SHARED((NPAD,), jnp.float32),
          pltpu.VMEM((NCH, CHUNK), jnp.int32),
          pltpu.VMEM((NCH, CHUNK), jnp.float32),
          pltpu.VMEM((CHUNK,), jnp.float32),
          pltpu.VMEM((RPT,), jnp.float32),
      ],
  )
  def _deg_kernel(cols0, w0, cols1, w1, cols2, w2, degp,
                  d0, d1, d2, cols_a, w_a, zbuf, vbuf):
    c = lax.axis_index("c")
    s = lax.axis_index("s")
    wid = s * NC + c
    z = jnp.zeros((16,), jnp.float32)
    for j in range(CHUNK // 16):
        zbuf[pl.ds(j * 16, 16)] = z
    for dk in (d0, d1, d2):
        for j in range(RPT // CHUNK):
            pltpu.sync_copy(zbuf, dk.at[pl.ds(s * RPT + j * CHUNK, CHUNK)])
    plsc.subcore_barrier()
    for cols, w, dk in ((cols0, w0, d0), (cols1, w1, d1), (cols2, w2, d2)):
        pltpu.sync_copy(cols.at[pl.ds(wid * NCH, NCH)], cols_a)
        pltpu.sync_copy(w.at[pl.ds(wid * NCH, NCH)], w_a)

        def chunk_body(ci, _, dk=dk):
            pltpu.sync_copy(w_a.at[ci], dk.at[cols_a.at[ci]], add=True)
            return ()
        lax.fori_loop(0, NCH, chunk_body, ())
    plsc.subcore_barrier()
    for k, dk in enumerate((d0, d1, d2)):
        pltpu.sync_copy(dk.at[pl.ds(s * RPT, RPT)], vbuf)
        pltpu.sync_copy(vbuf, degp.at[pl.ds((c * 3 + k) * NPAD + s * RPT, RPT)])

  @functools.partial(
      pl.kernel,
      out_type=jax.ShapeDtypeStruct((3 * NW * NCH, CHUNK), jnp.float32),
      mesh=mesh,
      compiler_params=_sc_params,
      scratch_types=[
          pltpu.VMEM((NPAD,), jnp.float32),
          pltpu.VMEM((NCH, CHUNK), jnp.int32),
          pltpu.VMEM((NCH, CHUNK), jnp.int32),
          pltpu.VMEM((NCH, CHUNK), jnp.float32),
          pltpu.VMEM((NCH, CHUNK), jnp.float32),
      ],
  )
  def _norm_kernel(dis, r0, c0, w0, r1, c1, w1, r2, c2, w2,
                   normh, dis_v, rows_a, cols_a, w_a, norm_a):
    c = lax.axis_index("c")
    s = lax.axis_index("s")
    wid = s * NC + c
    for k, (rows, cols, w) in enumerate(((r0, c0, w0), (r1, c1, w1), (r2, c2, w2))):
        pltpu.sync_copy(dis.at[pl.ds(k * NPAD, NPAD)], dis_v)
        pltpu.sync_copy(rows.at[pl.ds(wid * NCH, NCH)], rows_a)
        pltpu.sync_copy(cols.at[pl.ds(wid * NCH, NCH)], cols_a)
        pltpu.sync_copy(w.at[pl.ds(wid * NCH, NCH)], w_a)

        @plsc.parallel_loop(0, NCH, unroll=2)
        def norm_body(j):
            for g in range(CHUNK // 16):
                r16 = rows_a[j, pl.ds(g * 16, 16)]
                c16 = cols_a[j, pl.ds(g * 16, 16)]
                dr = plsc.load_gather(dis_v, [r16])
                dc = plsc.load_gather(dis_v, [c16])
                norm_a[j, pl.ds(g * 16, 16)] = dr * w_a[j, pl.ds(g * 16, 16)] * dc
        pltpu.sync_copy(norm_a, normh.at[pl.ds((k * NW + wid) * NCH, NCH)])

  @functools.partial(
      pl.kernel,
      out_type=jax.ShapeDtypeStruct((2, NPAD, H), jnp.float32),
      mesh=mesh,
      compiler_params=_sc_params,
      scratch_types=[
          pltpu.VMEM_SHARED((NPAD, H), jnp.float32),
          pltpu.VMEM((CHUNK, H), jnp.float32),
          pltpu.VMEM((CHUNK, H), jnp.float32),
          pltpu.VMEM((G, CHUNK), jnp.int32),
          pltpu.VMEM((G, CHUNK), jnp.int32),
          pltpu.VMEM((G, CHUNK), jnp.float32),
          pltpu.SemaphoreType.DMA,
          pltpu.SemaphoreType.DMA,
          pltpu.SemaphoreType.DMA,
          pltpu.SemaphoreType.DMA,
      ],
  )
  def _agg_kernel(xw, normh, r0, c0, r1, c1, r2, c2,
                  out, acc, bufx, bufy, rows_a, cols_a, norm_a,
                  gsx, gsy, ssx, ssy):
    c = lax.axis_index("c")
    s = lax.axis_index("s")
    wid = s * NC + c
    z = jnp.zeros((16,), jnp.float32)

    @plsc.parallel_loop(0, CHUNK, unroll=4)
    def _zero(i):
        for j in range(H // 16):
            bufx[i, pl.ds(j * 16, 16)] = z
    for j in range(RPT // CHUNK):
        pltpu.sync_copy(bufx, acc.at[pl.ds(s * RPT + j * CHUNK, CHUNK)])
    plsc.subcore_barrier()

    def scale(buf, ci):
        cibc = jnp.zeros((16,), jnp.int32) + ci

        @plsc.parallel_loop(0, CHUNK, unroll=4)
        def scale_body(e):
            nb = plsc.load_gather(
                norm_a, [cibc, jnp.zeros((16,), jnp.int32) + e])
            for j in range(H // 16):
                buf[e, pl.ds(j * 16, 16)] = buf[e, pl.ds(j * 16, 16)] * nb

    for k, (rows, cols) in enumerate(((r0, c0), (r1, c1), (r2, c2))):
        def group_body(g, _, rows=rows, cols=cols, k=k):
            base = wid * NCH + g * G
            pltpu.sync_copy(rows.at[pl.ds(base, G)], rows_a)
            pltpu.sync_copy(cols.at[pl.ds(base, G)], cols_a)
            pltpu.sync_copy(normh.at[pl.ds(k * NW * NCH + base, G)], norm_a)
            pltpu.async_copy(xw.at[rows_a.at[0]], bufx, gsx)
            pltpu.async_copy(xw.at[rows_a.at[1]], bufy, gsy)

            def pair_body(i, _):
                ci = 2 * i
                pltpu.make_async_copy(xw.at[rows_a.at[ci]], bufx, gsx).wait()
                scale(bufx, ci)
                pltpu.make_async_copy(xw.at[rows_a.at[ci + 1]], bufy, gsy).wait()
                pltpu.async_copy(bufx, acc.at[cols_a.at[ci]], ssx, add=True)
                scale(bufy, ci + 1)
                pltpu.async_copy(bufy, acc.at[cols_a.at[ci + 1]], ssy, add=True)

                @pl.when(i < G // 2 - 1)
                def _prefetch():
                    pltpu.make_async_copy(bufx, acc.at[cols_a.at[ci]], ssx).wait()
                    pltpu.async_copy(xw.at[rows_a.at[ci + 2]], bufx, gsx)
                    pltpu.make_async_copy(bufy, acc.at[cols_a.at[ci + 1]], ssy).wait()
                    pltpu.async_copy(xw.at[rows_a.at[ci + 3]], bufy, gsy)
                return ()
            lax.fori_loop(0, G // 2, pair_body, ())
            pltpu.make_async_copy(bufx, acc.at[cols_a.at[G - 2]], ssx).wait()
            pltpu.make_async_copy(bufy, acc.at[cols_a.at[G - 1]], ssy).wait()
            return ()
        lax.fori_loop(0, NCH // G, group_body, ())

    plsc.subcore_barrier()
    for j in range(RPT // CHUNK):
        pltpu.sync_copy(acc.at[pl.ds(s * RPT + j * CHUNK, CHUNK)],
                        out.at[c, pl.ds(s * RPT + j * CHUNK, CHUNK)])

  return _deg_kernel, _norm_kernel, _agg_kernel


def _mm_body(x_ref, w_ref, o_ref):
    o_ref[...] = jnp.dot(x_ref[...], w_ref[...],
                         preferred_element_type=jnp.float32)


def _dis_body(degp_ref, dis_ref):
    deg = degp_ref[0:3, :] + degp_ref[3:6, :]
    safe = jnp.where(deg > 0, deg, 1.0)
    dis_ref[...] = jnp.where(deg > 0, lax.rsqrt(safe), 0.0)


def _final_body(p0_ref, p1_ref, b_ref, o_ref):
    s = p0_ref[0] + p1_ref[0] + 3.0 * b_ref[...]
    o_ref[...] = jnp.maximum(s, 0.0)


def _pad_edges(ei, ew):
    pad = EPT - E
    padcols = (N + (jnp.arange(pad, dtype=jnp.int32) % 16)).astype(jnp.int32)
    rows = jnp.concatenate([ei[0], jnp.zeros((pad,), jnp.int32)])
    cols = jnp.concatenate([ei[1], padcols])
    w = jnp.concatenate([ew, jnp.zeros((pad,), jnp.float32)])
    return (rows.reshape(NW * NCH, CHUNK), cols.reshape(NW * NCH, CHUNK),
            w.reshape(NW * NCH, CHUNK))


@jax.jit
def kernel(x, edge_index0, edge_weight0, edge_index1, edge_weight1,
           edge_index2, edge_weight2, W, b):
    r0, c0, w0 = _pad_edges(edge_index0, edge_weight0)
    r1, c1, w1 = _pad_edges(edge_index1, edge_weight1)
    r2, c2, w2 = _pad_edges(edge_index2, edge_weight2)

    deg_kernel, norm_kernel, agg_kernel = _sc_kernels()
    degp = deg_kernel(c0, w0, c1, w1, c2, w2).reshape(6, NPAD)

    xw = pl.pallas_call(
        _mm_body,
        out_shape=jax.ShapeDtypeStruct((N, H), jnp.float32),
        grid=(10,),
        in_specs=[pl.BlockSpec((1000, D), lambda i: (i, 0)),
                  pl.BlockSpec((D, H), lambda i: (0, 0))],
        out_specs=pl.BlockSpec((1000, H), lambda i: (i, 0)),
    )(x, W)

    dis = pl.pallas_call(
        _dis_body,
        out_shape=jax.ShapeDtypeStruct((3, NPAD), jnp.float32),
        in_specs=[pl.BlockSpec((6, NPAD), lambda: (0, 0))],
        out_specs=pl.BlockSpec((3, NPAD), lambda: (0, 0)),
    )(degp)

    normh = norm_kernel(dis.reshape(3 * NPAD), r0, c0, w0, r1, c1, w1, r2, c2, w2)
    p = agg_kernel(xw, normh, r0, c0, r1, c1, r2, c2)

    b2 = b.reshape(1, H)
    out = pl.pallas_call(
        _final_body,
        out_shape=jax.ShapeDtypeStruct((N, H), jnp.float32),
        grid=(10,),
        in_specs=[pl.BlockSpec((1, 1000, H), lambda i: (0, i, 0)),
                  pl.BlockSpec((1, 1000, H), lambda i: (1, i, 0)),
                  pl.BlockSpec((1, H), lambda i: (0, 0))],
        out_specs=pl.BlockSpec((1000, H), lambda i: (i, 0)),
    )(p, p, b2)
    return out


# speedup vs baseline: 31.2418x; 2.7712x over previous
import functools
import jax
import jax.numpy as jnp
from jax import lax
from jax.experimental import pallas as pl
from jax.experimental.pallas import tpu as pltpu
from jax.experimental.pallas import tpu_sc as plsc

N = 10000
E = 320000
D = 128
H = 128

NC = 2
NS = 16
NW = NC * NS

CHUNK = 128
NCH = 80
EW = NCH * CHUNK
EPT = NW * EW
NPAD = 10240
RPT = NPAD // NS
G = 16

_sc_params = pltpu.CompilerParams(needs_layout_passes=False)


@functools.cache
def _sc_kernels():
  mesh = plsc.VectorSubcoreMesh(core_axis_name="c", subcore_axis_name="s",
                                num_cores=NC, num_subcores=NS)

  @functools.partial(
      pl.kernel,
      out_type=jax.ShapeDtypeStruct((6 * NPAD,), jnp.float32),
      mesh=mesh,
      compiler_params=_sc_params,
      scratch_types=[
          pltpu.VMEM_SHARED((NPAD,), jnp.float32),
          pltpu.VMEM_SHARED((NPAD,), jnp.float32),
          pltpu.VMEM_SHARED((NPAD,), jnp.float32),
          pltpu.VMEM((NCH, CHUNK), jnp.int32),
          pltpu.VMEM((NCH, CHUNK), jnp.float32),
          pltpu.VMEM((CHUNK,), jnp.float32),
          pltpu.VMEM((RPT,), jnp.float32),
      ],
  )
  def _deg_kernel(cols0, w0, cols1, w1, cols2, w2, degp,
                  d0, d1, d2, cols_a, w_a, zbuf, vbuf):
    c = lax.axis_index("c")
    s = lax.axis_index("s")
    wid = s * NC + c
    z = jnp.zeros((16,), jnp.float32)
    for j in range(CHUNK // 16):
        zbuf[pl.ds(j * 16, 16)] = z
    for dk in (d0, d1, d2):
        for j in range(RPT // CHUNK):
            pltpu.sync_copy(zbuf, dk.at[pl.ds(s * RPT + j * CHUNK, CHUNK)])
    plsc.subcore_barrier()
    for cols, w, dk in ((cols0, w0, d0), (cols1, w1, d1), (cols2, w2, d2)):
        pltpu.sync_copy(cols.at[pl.ds(wid * NCH, NCH)], cols_a)
        pltpu.sync_copy(w.at[pl.ds(wid * NCH, NCH)], w_a)

        def chunk_body(ci, _, dk=dk):
            pltpu.sync_copy(w_a.at[ci], dk.at[cols_a.at[ci]], add=True)
            return ()
        lax.fori_loop(0, NCH, chunk_body, ())
    plsc.subcore_barrier()
    for k, dk in enumerate((d0, d1, d2)):
        pltpu.sync_copy(dk.at[pl.ds(s * RPT, RPT)], vbuf)
        pltpu.sync_copy(vbuf, degp.at[pl.ds((c * 3 + k) * NPAD + s * RPT, RPT)])

  @functools.partial(
      pl.kernel,
      out_type=jax.ShapeDtypeStruct((3 * NW * NCH, CHUNK), jnp.float32),
      mesh=mesh,
      compiler_params=_sc_params,
      scratch_types=[
          pltpu.VMEM((NPAD,), jnp.float32),
          pltpu.VMEM((NCH, CHUNK), jnp.int32),
          pltpu.VMEM((NCH, CHUNK), jnp.int32),
          pltpu.VMEM((NCH, CHUNK), jnp.float32),
          pltpu.VMEM((NCH, CHUNK), jnp.float32),
      ],
  )
  def _norm_kernel(dis, r0, c0, w0, r1, c1, w1, r2, c2, w2,
                   normh, dis_v, rows_a, cols_a, w_a, norm_a):
    c = lax.axis_index("c")
    s = lax.axis_index("s")
    wid = s * NC + c
    for k, (rows, cols, w) in enumerate(((r0, c0, w0), (r1, c1, w1), (r2, c2, w2))):
        pltpu.sync_copy(dis.at[pl.ds(k * NPAD, NPAD)], dis_v)
        pltpu.sync_copy(rows.at[pl.ds(wid * NCH, NCH)], rows_a)
        pltpu.sync_copy(cols.at[pl.ds(wid * NCH, NCH)], cols_a)
        pltpu.sync_copy(w.at[pl.ds(wid * NCH, NCH)], w_a)

        @plsc.parallel_loop(0, NCH, unroll=2)
        def norm_body(j):
            for g in range(CHUNK // 16):
                r16 = rows_a[j, pl.ds(g * 16, 16)]
                c16 = cols_a[j, pl.ds(g * 16, 16)]
                dr = plsc.load_gather(dis_v, [r16])
                dc = plsc.load_gather(dis_v, [c16])
                norm_a[j, pl.ds(g * 16, 16)] = dr * w_a[j, pl.ds(g * 16, 16)] * dc
        pltpu.sync_copy(norm_a, normh.at[pl.ds((k * NW + wid) * NCH, NCH)])

  @functools.partial(
      pl.kernel,
      out_type=jax.ShapeDtypeStruct((2, NPAD, H), jnp.float32),
      mesh=mesh,
      compiler_params=_sc_params,
      scratch_types=[
          pltpu.VMEM_SHARED((NPAD, H), jnp.float32),
          pltpu.VMEM((CHUNK, H), jnp.float32),
          pltpu.VMEM((CHUNK, H), jnp.float32),
          pltpu.VMEM((G, CHUNK), jnp.int32),
          pltpu.VMEM((G, CHUNK), jnp.int32),
          pltpu.VMEM((G, CHUNK), jnp.float32),
          pltpu.SemaphoreType.DMA,
          pltpu.SemaphoreType.DMA,
          pltpu.SemaphoreType.DMA,
          pltpu.SemaphoreType.DMA,
      ],
  )
  def _agg_kernel(xw, normh, r0, c0, r1, c1, r2, c2,
                  out, acc, bufx, bufy, rows_a, cols_a, norm_a,
                  gsx, gsy, ssx, ssy):
    c = lax.axis_index("c")
    s = lax.axis_index("s")
    wid = s * NC + c
    z = jnp.zeros((16,), jnp.float32)

    @plsc.parallel_loop(0, CHUNK, unroll=4)
    def _zero(i):
        for j in range(H // 16):
            bufx[i, pl.ds(j * 16, 16)] = z
    for j in range(RPT // CHUNK):
        pltpu.sync_copy(bufx, acc.at[pl.ds(s * RPT + j * CHUNK, CHUNK)])
    plsc.subcore_barrier()

    def scale(buf, ci):
        cibc = jnp.zeros((16,), jnp.int32) + ci

        @plsc.parallel_loop(0, CHUNK, unroll=4)
        def scale_body(e):
            nb = plsc.load_gather(
                norm_a, [cibc, jnp.zeros((16,), jnp.int32) + e])
            for j in range(H // 16):
                buf[e, pl.ds(j * 16, 16)] = buf[e, pl.ds(j * 16, 16)] * nb

    for k, (rows, cols) in enumerate(((r0, c0), (r1, c1), (r2, c2))):
        def group_body(g, _, rows=rows, cols=cols, k=k):
            base = wid * NCH + g * G
            pltpu.sync_copy(rows.at[pl.ds(base, G)], rows_a)
            pltpu.sync_copy(cols.at[pl.ds(base, G)], cols_a)
            pltpu.sync_copy(normh.at[pl.ds(k * NW * NCH + base, G)], norm_a)
            pltpu.async_copy(xw.at[rows_a.at[0]], bufx, gsx)
            pltpu.async_copy(xw.at[rows_a.at[1]], bufy, gsy)

            def pair_body(i, _):
                ci = 2 * i
                pltpu.make_async_copy(xw.at[rows_a.at[ci]], bufx, gsx).wait()
                scale(bufx, ci)
                pltpu.make_async_copy(xw.at[rows_a.at[ci + 1]], bufy, gsy).wait()
                pltpu.async_copy(bufx, acc.at[cols_a.at[ci]], ssx, add=True)
                scale(bufy, ci + 1)
                pltpu.async_copy(bufy, acc.at[cols_a.at[ci + 1]], ssy, add=True)

                @pl.when(i < G // 2 - 1)
                def _prefetch():
                    pltpu.make_async_copy(bufx, acc.at[cols_a.at[ci]], ssx).wait()
                    pltpu.async_copy(xw.at[rows_a.at[ci + 2]], bufx, gsx)
                    pltpu.make_async_copy(bufy, acc.at[cols_a.at[ci + 1]], ssy).wait()
                    pltpu.async_copy(xw.at[rows_a.at[ci + 3]], bufy, gsy)
                return ()
            lax.fori_loop(0, G // 2, pair_body, ())
            pltpu.make_async_copy(bufx, acc.at[cols_a.at[G - 2]], ssx).wait()
            pltpu.make_async_copy(bufy, acc.at[cols_a.at[G - 1]], ssy).wait()
            return ()
        lax.fori_loop(0, NCH // G, group_body, ())

    plsc.subcore_barrier()
    for j in range(RPT // CHUNK):
        pltpu.sync_copy(acc.at[pl.ds(s * RPT + j * CHUNK, CHUNK)],
                        out.at[c, pl.ds(s * RPT + j * CHUNK, CHUNK)])

  return _deg_kernel, _norm_kernel, _agg_kernel


def _mm_body(x_ref, w_ref, o_ref):
    o_ref[...] = jnp.dot(x_ref[...], w_ref[...],
                         preferred_element_type=jnp.float32)


def _dis_body(degp_ref, dis_ref):
    deg = degp_ref[0:3, :] + degp_ref[3:6, :]
    safe = jnp.where(deg > 0, deg, 1.0)
    dis_ref[...] = jnp.where(deg > 0, lax.rsqrt(safe), 0.0)


def _final_body(p0_ref, p1_ref, b_ref, o_ref):
    s = p0_ref[0] + p1_ref[0] + 3.0 * b_ref[...]
    o_ref[...] = jnp.maximum(s, 0.0)


def _pad_edges(ei, ew):
    pad = EPT - E
    idx = jnp.arange(pad, dtype=jnp.int32)
    padrows = (idx * 37) % N
    padcols = N + (idx % (NPAD - N))
    rows = jnp.concatenate([ei[0], padrows])
    cols = jnp.concatenate([ei[1], padcols])
    w = jnp.concatenate([ew, jnp.zeros((pad,), jnp.float32)])
    return (rows.reshape(NW * NCH, CHUNK), cols.reshape(NW * NCH, CHUNK),
            w.reshape(NW * NCH, CHUNK))


@jax.jit
def kernel(x, edge_index0, edge_weight0, edge_index1, edge_weight1,
           edge_index2, edge_weight2, W, b):
    r0, c0, w0 = _pad_edges(edge_index0, edge_weight0)
    r1, c1, w1 = _pad_edges(edge_index1, edge_weight1)
    r2, c2, w2 = _pad_edges(edge_index2, edge_weight2)

    deg_kernel, norm_kernel, agg_kernel = _sc_kernels()
    degp = deg_kernel(c0, w0, c1, w1, c2, w2).reshape(6, NPAD)

    xw = pl.pallas_call(
        _mm_body,
        out_shape=jax.ShapeDtypeStruct((N, H), jnp.float32),
        grid=(10,),
        in_specs=[pl.BlockSpec((1000, D), lambda i: (i, 0)),
                  pl.BlockSpec((D, H), lambda i: (0, 0))],
        out_specs=pl.BlockSpec((1000, H), lambda i: (i, 0)),
    )(x, W)

    dis = pl.pallas_call(
        _dis_body,
        out_shape=jax.ShapeDtypeStruct((3, NPAD), jnp.float32),
        in_specs=[pl.BlockSpec((6, NPAD), lambda: (0, 0))],
        out_specs=pl.BlockSpec((3, NPAD), lambda: (0, 0)),
    )(degp)

    normh = norm_kernel(dis.reshape(3 * NPAD), r0, c0, w0, r1, c1, w1, r2, c2, w2)
    p = agg_kernel(xw, normh, r0, c0, r1, c1, r2, c2)

    b2 = b.reshape(1, H)
    out = pl.pallas_call(
        _final_body,
        out_shape=jax.ShapeDtypeStruct((N, H), jnp.float32),
        grid=(10,),
        in_specs=[pl.BlockSpec((1, 1000, H), lambda i: (0, i, 0)),
                  pl.BlockSpec((1, 1000, H), lambda i: (1, i, 0)),
                  pl.BlockSpec((1, H), lambda i: (0, 0))],
        out_specs=pl.BlockSpec((1000, H), lambda i: (i, 0)),
    )(p, p, b2)
    return out
